# Initial kernel scaffold; baseline (speedup 1.0000x reference)
import jax
import jax.numpy as jnp
from jax.experimental import pallas as pl


def kernel(receptor_node_features, ligand_node_features, receptor_edge_index, ligand_edge_index, params):
    raise NotImplementedError("write your pallas kernel here")


# speedup vs baseline: 3.0123x; 3.0123x over previous
import functools

import jax
import jax.numpy as jnp
from jax import lax
from jax.experimental import pallas as pl
from jax.experimental.pallas import tpu as pltpu
from jax.experimental.pallas import tpu_sc as plsc

HID = 128
NUM_LAYERS = 3
N_NODES = 10000
N_PAD = 10112
CHUNK = 128
NW = 32
LN_EPS = 1e-5


GRP = 8


def _make_segsum(n_super: int):
    rows_per_sub = N_PAD // 16

    mesh = plsc.VectorSubcoreMesh(core_axis_name="c", subcore_axis_name="s")

    @functools.partial(
        pl.kernel,
        out_type=jax.ShapeDtypeStruct((2, N_PAD, HID), jnp.float32),
        mesh=mesh,
        scratch_types=[
            pltpu.VMEM((GRP, CHUNK), jnp.int32),
            pltpu.VMEM((GRP, CHUNK), jnp.int32),
            pltpu.VMEM((CHUNK, HID), jnp.float32),
            pltpu.VMEM_SHARED((N_PAD, HID), jnp.float32),
            pltpu.SemaphoreType.DMA,
        ],
    )
    def segsum(xt_hbm, src_hbm, dst_hbm, zeros_hbm, out_hbm,
               src_v, dst_v, rows_v, acc_sh, sem):
        c = lax.axis_index("c")
        s = lax.axis_index("s")
        wid = s * 2 + c

        pltpu.sync_copy(zeros_hbm.at[pl.ds(s * rows_per_sub, rows_per_sub)],
                        acc_sh.at[pl.ds(s * rows_per_sub, rows_per_sub)])
        plsc.subcore_barrier()

        def body(t, carry):
            pltpu.sync_copy(src_hbm.at[wid, t], src_v)
            pltpu.sync_copy(dst_hbm.at[wid, t], dst_v)
            for g in range(GRP):
                pltpu.async_copy(xt_hbm.at[src_v.at[g]], rows_v, sem).wait()
                pltpu.sync_copy(rows_v, acc_sh.at[dst_v.at[g]], add=True)
            return carry

        lax.fori_loop(0, n_super, body, 0)
        plsc.subcore_barrier()
        pltpu.sync_copy(acc_sh.at[pl.ds(s * rows_per_sub, rows_per_sub)],
                        out_hbm.at[c, pl.ds(s * rows_per_sub, rows_per_sub)])

    return segsum


def _pad_edges(ei, e_pad):
    e = ei.shape[1]
    src = jnp.concatenate(
        [ei[0].astype(jnp.int32), jnp.zeros((e_pad - e,), jnp.int32)])
    dst = jnp.concatenate(
        [ei[1].astype(jnp.int32),
         jnp.full((e_pad - e,), N_PAD - 1, jnp.int32)])
    shape = (NW, e_pad // (NW * GRP * CHUNK), GRP, CHUNK)
    return src.reshape(shape), dst.reshape(shape)


def _input_proj_body(nf_ref, inw_ref, inb_ref, w0_ref, b0_ref, x_ref, xt_ref):
    x = jax.nn.relu(
        jnp.dot(nf_ref[...], inw_ref[...], preferred_element_type=jnp.float32)
        + inb_ref[...])
    x_ref[...] = x
    xt_ref[...] = jnp.dot(x, w0_ref[...],
                          preferred_element_type=jnp.float32) + b0_ref[...]


def _input_proj(nf_pad, inw_pad, inb, w0, b0):
    return pl.pallas_call(
        _input_proj_body,
        out_shape=(jax.ShapeDtypeStruct((N_NODES, HID), jnp.float32),
                   jax.ShapeDtypeStruct((N_NODES, HID), jnp.float32)),
    )(nf_pad, inw_pad, inb, w0, b0)


def _layer_update_body(has_next, xt_ref, s2_ref, g_ref, be_ref, x_ref,
                       wn_ref, bn_ref, xn_ref, xtn_ref=None):
    agg = xt_ref[...] + s2_ref[0, :N_NODES, :] + s2_ref[1, :N_NODES, :]
    mu = jnp.mean(agg, axis=-1, keepdims=True)
    var = jnp.mean((agg - mu) ** 2, axis=-1, keepdims=True)
    normed = (agg - mu) / jnp.sqrt(var + LN_EPS) * g_ref[...] + be_ref[...]
    xn = jax.nn.relu(normed) + x_ref[...]
    xn_ref[...] = xn
    if has_next:
        xtn_ref[...] = jnp.dot(xn, wn_ref[...],
                               preferred_element_type=jnp.float32) + bn_ref[...]


def _layer_update(xt, s2, g, be, x, wn, bn, has_next):
    if has_next:
        out_shape = (jax.ShapeDtypeStruct((N_NODES, HID), jnp.float32),
                     jax.ShapeDtypeStruct((N_NODES, HID), jnp.float32))
    else:
        out_shape = (jax.ShapeDtypeStruct((N_NODES, HID), jnp.float32),)
    res = pl.pallas_call(
        functools.partial(_layer_update_body, has_next),
        out_shape=out_shape,
    )(xt, s2, g, be, x, wn, bn)
    return res if has_next else (res[0], None)


def _head_body(xr_ref, xl_ref, pwr_ref, pbr_ref, pwl_ref, pbl_ref,
               c0w_ref, c0b_ref, c2w_ref, c2b_ref,
               rotw_ref, rotb_ref, trw_ref, trb_ref, cfw_ref, cfb_ref,
               out_ref):
    mr = jnp.mean(xr_ref[...], axis=0, keepdims=True)
    ml = jnp.mean(xl_ref[...], axis=0, keepdims=True)
    re = jnp.dot(mr, pwr_ref[...], preferred_element_type=jnp.float32) + pbr_ref[...]
    le = jnp.dot(ml, pwl_ref[...], preferred_element_type=jnp.float32) + pbl_ref[...]
    h = jnp.concatenate([re, le], axis=1)
    h = jax.nn.relu(
        jnp.dot(h, c0w_ref[...], preferred_element_type=jnp.float32) + c0b_ref[...])
    h = jax.nn.relu(
        jnp.dot(h, c2w_ref[...], preferred_element_type=jnp.float32) + c2b_ref[...])
    rot = jnp.dot(h, rotw_ref[...], preferred_element_type=jnp.float32) + rotb_ref[...]
    tr = jnp.dot(h, trw_ref[...], preferred_element_type=jnp.float32) + trb_ref[...]
    cf = jax.nn.sigmoid(
        jnp.dot(h, cfw_ref[...], preferred_element_type=jnp.float32) + cfb_ref[...])
    out_ref[...] = jnp.concatenate(
        [rot, tr, cf, jnp.zeros((5, HID), jnp.float32)], axis=0)


def _head(xr, xl, args):
    return pl.pallas_call(
        _head_body,
        out_shape=jax.ShapeDtypeStruct((8, HID), jnp.float32),
    )(xr, xl, *args)


def _pad_cols(w, cols=HID):
    return jnp.pad(w, ((0, 0), (0, cols - w.shape[1])))


def _row(v):
    return v.reshape(1, -1)


def _encoder(nf, src2d, dst2d, segsum, p, zeros_pad):
    nf_pad = jnp.pad(nf, ((0, 0), (0, HID - nf.shape[1])))
    inw_pad = jnp.pad(p['in_W'], ((0, HID - p['in_W'].shape[0]), (0, 0)))
    x, xt = _input_proj(nf_pad, inw_pad, _row(p['in_b']),
                        p['conv0_W'], _row(p['conv0_b']))
    for i in range(NUM_LAYERS):
        s2 = segsum(jnp.pad(xt, ((0, N_PAD - N_NODES), (0, 0))),
                    src2d, dst2d, zeros_pad)
        has_next = i + 1 < NUM_LAYERS
        wn = p['conv%d_W' % (i + 1)] if has_next else p['conv0_W']
        bn = p['conv%d_b' % (i + 1)] if has_next else p['conv0_b']
        x, xt = _layer_update(xt, s2,
                              _row(p['conv%d_g' % i]), _row(p['conv%d_be' % i]),
                              x, wn, _row(bn), has_next)
    return x


def kernel(receptor_node_features, ligand_node_features,
           receptor_edge_index, ligand_edge_index, params):
    e_r = receptor_edge_index.shape[1]
    e_l = ligand_edge_index.shape[1]
    block = NW * GRP * CHUNK
    e_r_pad = ((e_r + block - 1) // block) * block
    e_l_pad = ((e_l + block - 1) // block) * block

    src_r, dst_r = _pad_edges(receptor_edge_index, e_r_pad)
    src_l, dst_l = _pad_edges(ligand_edge_index, e_l_pad)
    zeros_pad = jnp.zeros((N_PAD, HID), jnp.float32)

    seg_r = _make_segsum(e_r_pad // block)
    seg_l = _make_segsum(e_l_pad // block)
    if e_r_pad == e_l_pad:
        seg_l = seg_r

    xr = _encoder(receptor_node_features, src_r, dst_r, seg_r,
                  params['rec'], zeros_pad)
    xl = _encoder(ligand_node_features, src_l, dst_l, seg_l,
                  params['lig'], zeros_pad)

    head_args = (
        params['rec']['pool_W'], _row(params['rec']['pool_b']),
        params['lig']['pool_W'], _row(params['lig']['pool_b']),
        params['c0_W'], _row(params['c0_b']),
        params['c2_W'], _row(params['c2_b']),
        _pad_cols(params['rot_W']), _row(_pad_cols(_row(params['rot_b']))[0]),
        _pad_cols(params['tr_W']), _row(_pad_cols(_row(params['tr_b']))[0]),
        _pad_cols(params['conf_W']), _row(_pad_cols(_row(params['conf_b']))[0]),
    )
    out = _head(xr, xl, head_args)
    rotation = out[0, :3]
    translation = out[1, :3]
    confidence = out[2, :1]
    return rotation, translation, confidence


# speedup vs baseline: 3.4852x; 1.1570x over previous
import functools

import jax
import jax.numpy as jnp
from jax import lax
from jax.experimental import pallas as pl
from jax.experimental.pallas import tpu as pltpu
from jax.experimental.pallas import tpu_sc as plsc

HID = 128
NUM_LAYERS = 3
N_NODES = 10000
N_PAD = 10112
CHUNK = 128
NW = 32
LN_EPS = 1e-5


GRP = 8


def _make_segsum(n_super: int):
    rows_per_sub = N_PAD // 16

    mesh = plsc.VectorSubcoreMesh(core_axis_name="c", subcore_axis_name="s")

    n_w = n_super * GRP

    @functools.partial(
        pl.kernel,
        out_type=jax.ShapeDtypeStruct((2, N_PAD, HID), jnp.float32),
        mesh=mesh,
        scratch_types=[
            pltpu.VMEM((2, GRP, CHUNK), jnp.int32),
            pltpu.VMEM((2, GRP, CHUNK), jnp.int32),
            pltpu.VMEM((CHUNK, HID), jnp.float32),
            pltpu.VMEM((CHUNK, HID), jnp.float32),
            pltpu.VMEM_SHARED((N_PAD, HID), jnp.float32),
            pltpu.SemaphoreType.DMA,
            pltpu.SemaphoreType.DMA,
        ],
    )
    def segsum(xt_hbm, src_hbm, dst_hbm, zeros_hbm, out_hbm,
               src_i, dst_i, rows_a, rows_b, acc_sh, sem_a, sem_b):
        c = lax.axis_index("c")
        s = lax.axis_index("s")
        wid = s * 2 + c

        def stage(t):
            p = lax.rem(t, 2)
            pltpu.sync_copy(src_hbm.at[wid, t], src_i.at[p])
            pltpu.sync_copy(dst_hbm.at[wid, t], dst_i.at[p])

        def src_row(j):
            t = lax.div(j, GRP)
            return src_i.at[lax.rem(t, 2), lax.rem(j, GRP)]

        def dst_row(j):
            t = lax.div(j, GRP)
            return dst_i.at[lax.rem(t, 2), lax.rem(j, GRP)]

        pltpu.sync_copy(zeros_hbm.at[pl.ds(s * rows_per_sub, rows_per_sub)],
                        acc_sh.at[pl.ds(s * rows_per_sub, rows_per_sub)])
        stage(0)
        pltpu.async_copy(xt_hbm.at[src_i.at[0, 0]], rows_a, sem_a)
        plsc.subcore_barrier()

        def body(j2, carry):
            ja = 2 * j2
            jb = ja + 1
            jc = ja + 2
            pltpu.async_copy(xt_hbm.at[src_row(jb)], rows_b, sem_b)
            pltpu.make_async_copy(xt_hbm.at[pl.ds(0, CHUNK)], rows_a, sem_a).wait()
            pltpu.sync_copy(rows_a, acc_sh.at[dst_row(ja)], add=True)

            @pl.when(jc < n_w)
            def _():
                @pl.when(lax.rem(jc, GRP) == 0)
                def _():
                    stage(lax.div(jc, GRP))
                pltpu.async_copy(xt_hbm.at[src_row(jc)], rows_a, sem_a)

            pltpu.make_async_copy(xt_hbm.at[pl.ds(0, CHUNK)], rows_b, sem_b).wait()
            pltpu.sync_copy(rows_b, acc_sh.at[dst_row(jb)], add=True)
            return carry

        lax.fori_loop(0, n_w // 2, body, 0)
        plsc.subcore_barrier()
        pltpu.sync_copy(acc_sh.at[pl.ds(s * rows_per_sub, rows_per_sub)],
                        out_hbm.at[c, pl.ds(s * rows_per_sub, rows_per_sub)])

    return segsum


def _pad_edges(ei, e_pad):
    e = ei.shape[1]
    src = jnp.concatenate(
        [ei[0].astype(jnp.int32), jnp.zeros((e_pad - e,), jnp.int32)])
    dst = jnp.concatenate(
        [ei[1].astype(jnp.int32),
         jnp.full((e_pad - e,), N_PAD - 1, jnp.int32)])
    shape = (NW, e_pad // (NW * GRP * CHUNK), GRP, CHUNK)
    return src.reshape(shape), dst.reshape(shape)


def _input_proj_body(nf_ref, inw_ref, inb_ref, w0_ref, b0_ref, x_ref, xt_ref):
    x = jax.nn.relu(
        jnp.dot(nf_ref[...], inw_ref[...], preferred_element_type=jnp.float32)
        + inb_ref[...])
    x_ref[...] = x
    xt_ref[...] = jnp.dot(x, w0_ref[...],
                          preferred_element_type=jnp.float32) + b0_ref[...]


def _input_proj(nf_pad, inw_pad, inb, w0, b0):
    return pl.pallas_call(
        _input_proj_body,
        out_shape=(jax.ShapeDtypeStruct((N_NODES, HID), jnp.float32),
                   jax.ShapeDtypeStruct((N_NODES, HID), jnp.float32)),
    )(nf_pad, inw_pad, inb, w0, b0)


def _layer_update_body(has_next, xt_ref, s2_ref, g_ref, be_ref, x_ref,
                       wn_ref, bn_ref, xn_ref, xtn_ref=None):
    agg = xt_ref[...] + s2_ref[0, :N_NODES, :] + s2_ref[1, :N_NODES, :]
    mu = jnp.mean(agg, axis=-1, keepdims=True)
    var = jnp.mean((agg - mu) ** 2, axis=-1, keepdims=True)
    normed = (agg - mu) / jnp.sqrt(var + LN_EPS) * g_ref[...] + be_ref[...]
    xn = jax.nn.relu(normed) + x_ref[...]
    xn_ref[...] = xn
    if has_next:
        xtn_ref[...] = jnp.dot(xn, wn_ref[...],
                               preferred_element_type=jnp.float32) + bn_ref[...]


def _layer_update(xt, s2, g, be, x, wn, bn, has_next):
    if has_next:
        out_shape = (jax.ShapeDtypeStruct((N_NODES, HID), jnp.float32),
                     jax.ShapeDtypeStruct((N_NODES, HID), jnp.float32))
    else:
        out_shape = (jax.ShapeDtypeStruct((N_NODES, HID), jnp.float32),)
    res = pl.pallas_call(
        functools.partial(_layer_update_body, has_next),
        out_shape=out_shape,
    )(xt, s2, g, be, x, wn, bn)
    return res if has_next else (res[0], None)


def _head_body(xr_ref, xl_ref, pwr_ref, pbr_ref, pwl_ref, pbl_ref,
               c0w_ref, c0b_ref, c2w_ref, c2b_ref,
               rotw_ref, rotb_ref, trw_ref, trb_ref, cfw_ref, cfb_ref,
               out_ref):
    mr = jnp.mean(xr_ref[...], axis=0, keepdims=True)
    ml = jnp.mean(xl_ref[...], axis=0, keepdims=True)
    re = jnp.dot(mr, pwr_ref[...], preferred_element_type=jnp.float32) + pbr_ref[...]
    le = jnp.dot(ml, pwl_ref[...], preferred_element_type=jnp.float32) + pbl_ref[...]
    h = jnp.concatenate([re, le], axis=1)
    h = jax.nn.relu(
        jnp.dot(h, c0w_ref[...], preferred_element_type=jnp.float32) + c0b_ref[...])
    h = jax.nn.relu(
        jnp.dot(h, c2w_ref[...], preferred_element_type=jnp.float32) + c2b_ref[...])
    rot = jnp.dot(h, rotw_ref[...], preferred_element_type=jnp.float32) + rotb_ref[...]
    tr = jnp.dot(h, trw_ref[...], preferred_element_type=jnp.float32) + trb_ref[...]
    cf = jax.nn.sigmoid(
        jnp.dot(h, cfw_ref[...], preferred_element_type=jnp.float32) + cfb_ref[...])
    out_ref[...] = jnp.concatenate(
        [rot, tr, cf, jnp.zeros((5, HID), jnp.float32)], axis=0)


def _head(xr, xl, args):
    return pl.pallas_call(
        _head_body,
        out_shape=jax.ShapeDtypeStruct((8, HID), jnp.float32),
    )(xr, xl, *args)


def _pad_cols(w, cols=HID):
    return jnp.pad(w, ((0, 0), (0, cols - w.shape[1])))


def _row(v):
    return v.reshape(1, -1)


def _encoder(nf, src2d, dst2d, segsum, p, zeros_pad):
    nf_pad = jnp.pad(nf, ((0, 0), (0, HID - nf.shape[1])))
    inw_pad = jnp.pad(p['in_W'], ((0, HID - p['in_W'].shape[0]), (0, 0)))
    x, xt = _input_proj(nf_pad, inw_pad, _row(p['in_b']),
                        p['conv0_W'], _row(p['conv0_b']))
    for i in range(NUM_LAYERS):
        s2 = segsum(jnp.pad(xt, ((0, N_PAD - N_NODES), (0, 0))),
                    src2d, dst2d, zeros_pad)
        has_next = i + 1 < NUM_LAYERS
        wn = p['conv%d_W' % (i + 1)] if has_next else p['conv0_W']
        bn = p['conv%d_b' % (i + 1)] if has_next else p['conv0_b']
        x, xt = _layer_update(xt, s2,
                              _row(p['conv%d_g' % i]), _row(p['conv%d_be' % i]),
                              x, wn, _row(bn), has_next)
    return x


def kernel(receptor_node_features, ligand_node_features,
           receptor_edge_index, ligand_edge_index, params):
    e_r = receptor_edge_index.shape[1]
    e_l = ligand_edge_index.shape[1]
    block = NW * GRP * CHUNK
    e_r_pad = ((e_r + block - 1) // block) * block
    e_l_pad = ((e_l + block - 1) // block) * block

    src_r, dst_r = _pad_edges(receptor_edge_index, e_r_pad)
    src_l, dst_l = _pad_edges(ligand_edge_index, e_l_pad)
    zeros_pad = jnp.zeros((N_PAD, HID), jnp.float32)

    seg_r = _make_segsum(e_r_pad // block)
    seg_l = _make_segsum(e_l_pad // block)
    if e_r_pad == e_l_pad:
        seg_l = seg_r

    xr = _encoder(receptor_node_features, src_r, dst_r, seg_r,
                  params['rec'], zeros_pad)
    xl = _encoder(ligand_node_features, src_l, dst_l, seg_l,
                  params['lig'], zeros_pad)

    head_args = (
        params['rec']['pool_W'], _row(params['rec']['pool_b']),
        params['lig']['pool_W'], _row(params['lig']['pool_b']),
        params['c0_W'], _row(params['c0_b']),
        params['c2_W'], _row(params['c2_b']),
        _pad_cols(params['rot_W']), _row(_pad_cols(_row(params['rot_b']))[0]),
        _pad_cols(params['tr_W']), _row(_pad_cols(_row(params['tr_b']))[0]),
        _pad_cols(params['conf_W']), _row(_pad_cols(_row(params['conf_b']))[0]),
    )
    out = _head(xr, xl, head_args)
    rotation = out[0, :3]
    translation = out[1, :3]
    confidence = out[2, :1]
    return rotation, translation, confidence
